# baseline (speedup 1.0000x reference)
import jax
import jax.numpy as jnp
from jax.experimental import pallas as pl
from jax.experimental.pallas import tpu as pltpu

NUM_FEATURES = 128
NUM_EDGES = 160000
NUM_NODES = 10000
ROWS = NUM_NODES + NUM_EDGES
BATCH = 4
N_IDX = 128

ZCHUNK = 8000
N_Z = NUM_EDGES // ZCHUNK


def _body(idx_ref, x_ref, out_ref, zbuf, nbuf, sem_z, sem_n):
    zbuf[...] = jnp.zeros_like(zbuf)
    for b in range(BATCH):
        for k in range(N_Z):
            pltpu.make_async_copy(
                zbuf, out_ref.at[pl.ds(b * ROWS + k * ZCHUNK, ZCHUNK)], sem_z.at[b]
            ).start()

    nbuf[...] = jnp.zeros_like(nbuf)

    def write_row(i, b):
        nbuf[b, pl.ds(idx_ref[i], 1), :] = x_ref[b, pl.ds(i, 1), :]
        return b

    for b in range(BATCH):
        jax.lax.fori_loop(0, N_IDX, write_row, b)
        pltpu.make_async_copy(
            nbuf.at[b], out_ref.at[pl.ds(b * ROWS + NUM_EDGES, NUM_NODES)], sem_n
        ).start()

    for b in range(BATCH):
        for k in range(N_Z):
            pltpu.make_async_copy(
                zbuf, out_ref.at[pl.ds(b * ROWS + k * ZCHUNK, ZCHUNK)], sem_z.at[b]
            ).wait()
        pltpu.make_async_copy(
            nbuf.at[b], out_ref.at[pl.ds(b * ROWS + NUM_EDGES, NUM_NODES)], sem_n
        ).wait()


def kernel(x, pivotal_nodes):
    bsz, n_idx, f = x.shape
    grid_spec = pltpu.PrefetchScalarGridSpec(
        num_scalar_prefetch=1,
        grid=(1,),
        in_specs=[pl.BlockSpec((bsz, n_idx, f), lambda i, idx: (0, 0, 0))],
        out_specs=pl.BlockSpec(memory_space=pl.ANY),
        scratch_shapes=[
            pltpu.VMEM((ZCHUNK, f), jnp.float32),
            pltpu.VMEM((bsz, NUM_NODES, f), jnp.float32),
            pltpu.SemaphoreType.DMA((BATCH,)),
            pltpu.SemaphoreType.DMA,
        ],
    )
    out_flat = pl.pallas_call(
        _body,
        grid_spec=grid_spec,
        out_shape=jax.ShapeDtypeStruct((bsz * ROWS, f), x.dtype),
    )(pivotal_nodes, x)
    return out_flat.reshape(bsz, ROWS, f)


# speedup vs baseline: 1.0047x; 1.0047x over previous
import jax
import jax.numpy as jnp
from jax.experimental import pallas as pl
from jax.experimental.pallas import tpu as pltpu

NUM_FEATURES = 128
NUM_EDGES = 160000
NUM_NODES = 10000
ROWS = NUM_NODES + NUM_EDGES
BATCH = 4
N_IDX = 128

ZCHUNK = 16000
N_Z = NUM_EDGES // ZCHUNK


def _body(idx_ref, x_ref, out_ref, zbuf, nbuf, sem_z, sem_n):
    zbuf[...] = jnp.zeros_like(zbuf)
    for b in range(BATCH):
        for k in range(N_Z):
            pltpu.make_async_copy(
                zbuf, out_ref.at[pl.ds(b * ROWS + k * ZCHUNK, ZCHUNK)], sem_z
            ).start()

    nbuf[...] = jnp.zeros_like(nbuf)

    def write_row(i, b):
        nbuf[b, pl.ds(idx_ref[i], 1), :] = x_ref[b, pl.ds(i, 1), :]
        return b

    for b in range(BATCH):
        jax.lax.fori_loop(0, N_IDX, write_row, b)
        pltpu.make_async_copy(
            nbuf.at[b], out_ref.at[pl.ds(b * ROWS + NUM_EDGES, NUM_NODES)], sem_n
        ).start()

    for b in range(BATCH):
        for k in range(N_Z):
            pltpu.make_async_copy(
                zbuf, out_ref.at[pl.ds(b * ROWS + k * ZCHUNK, ZCHUNK)], sem_z
            ).wait()
        pltpu.make_async_copy(
            nbuf.at[b], out_ref.at[pl.ds(b * ROWS + NUM_EDGES, NUM_NODES)], sem_n
        ).wait()


def kernel(x, pivotal_nodes):
    bsz, n_idx, f = x.shape
    grid_spec = pltpu.PrefetchScalarGridSpec(
        num_scalar_prefetch=1,
        grid=(1,),
        in_specs=[pl.BlockSpec((bsz, n_idx, f), lambda i, idx: (0, 0, 0))],
        out_specs=pl.BlockSpec(memory_space=pl.ANY),
        scratch_shapes=[
            pltpu.VMEM((ZCHUNK, f), jnp.float32),
            pltpu.VMEM((bsz, NUM_NODES, f), jnp.float32),
            pltpu.SemaphoreType.DMA,
            pltpu.SemaphoreType.DMA,
        ],
    )
    out_flat = pl.pallas_call(
        _body,
        grid_spec=grid_spec,
        out_shape=jax.ShapeDtypeStruct((bsz * ROWS, f), x.dtype),
    )(pivotal_nodes, x)
    return out_flat.reshape(bsz, ROWS, f)
